# Initial kernel scaffold; baseline (speedup 1.0000x reference)
import jax
import jax.numpy as jnp
from jax.experimental import pallas as pl


def kernel(x_SH, edge_index_SH, x_SS, edge_index_SS, x_HH, edge_index_HH, prescription, SH_table, W1, b1, W2, b2, W1h, b1h, W2h, b2h, mlp_W, mlp_b, bn_gamma, bn_beta):
    raise NotImplementedError("write your pallas kernel here")


# speedup vs baseline: 2.1592x; 2.1592x over previous
import functools

import jax
import jax.numpy as jnp
from jax import lax
from jax.experimental import pallas as pl
from jax.experimental.pallas import tpu as pltpu
from jax.experimental.pallas import tpu_sc as plsc

_NUM_U = 805
_NUM_I = 390
_N_SH = 1195
_D = 64
_K_U = 20
_K_I = 9

_VP = 1280
_AW = 128
_FO = 64
_NC = 2
_NS = 16
_EB = 128
_RB = 10
_E_PAD = _NC * _NS * _RB * _EB
_ROWS_PER_TILE = _VP // _NS


def _sc_agg_body(table_hbm, src_hbm, dst_hbm, zeros_hbm, out_hbm,
                 src_v, dst_v, rows0_v, rows1_v, stage_v, acc_sh,
                 sem0, sem1):
    c = lax.axis_index("c")
    s = lax.axis_index("s")
    wid = c * _NS + s
    row0 = s * _ROWS_PER_TILE
    bufs = (rows0_v, rows1_v)
    sems = (sem0, sem1)

    pltpu.sync_copy(zeros_hbm.at[pl.ds(row0, _ROWS_PER_TILE)], stage_v)
    pltpu.sync_copy(stage_v, acc_sh.at[pl.ds(row0, _ROWS_PER_TILE)])

    pltpu.sync_copy(src_hbm.at[wid], src_v)
    pltpu.sync_copy(dst_hbm.at[wid], dst_v)
    plsc.subcore_barrier()

    cps = [None] * _RB
    cps[0] = pltpu.async_copy(table_hbm.at[src_v.at[0]], bufs[0], sems[0])
    for r in range(_RB):
        if r + 1 < _RB:
            cps[r + 1] = pltpu.async_copy(
                table_hbm.at[src_v.at[r + 1]], bufs[(r + 1) % 2],
                sems[(r + 1) % 2])
        cps[r].wait()
        pltpu.sync_copy(bufs[r % 2], acc_sh.at[dst_v.at[r]], add=True)
    plsc.subcore_barrier()

    pltpu.sync_copy(acc_sh.at[pl.ds(row0, _ROWS_PER_TILE)], stage_v)
    pltpu.sync_copy(stage_v, out_hbm.at[c, pl.ds(row0, _ROWS_PER_TILE)])


@functools.cache
def _make_sc_agg():
    return pl.kernel(
        _sc_agg_body,
        out_type=jax.ShapeDtypeStruct((_NC, _VP, _AW), jnp.float32),
        mesh=plsc.VectorSubcoreMesh(core_axis_name="c", subcore_axis_name="s",
                                    num_cores=_NC, num_subcores=_NS),
        scratch_types=[
            pltpu.VMEM((_RB, _EB), jnp.int32),
            pltpu.VMEM((_RB, _EB), jnp.int32),
            pltpu.VMEM((_EB, _AW), jnp.float32),
            pltpu.VMEM((_EB, _AW), jnp.float32),
            pltpu.VMEM((_ROWS_PER_TILE, _AW), jnp.float32),
            pltpu.VMEM_SHARED((_VP, _AW), jnp.float32),
            pltpu.SemaphoreType.DMA,
            pltpu.SemaphoreType.DMA,
        ],
    )


def _sc_agg(table, src, dst, zeros):
    return _make_sc_agg()(table, src, dst, zeros)


def _aug(lin):
    row = lax.broadcasted_iota(jnp.int32, (_VP, 1), 0)
    valid = (row < _N_SH).astype(jnp.float32)
    lin = lin * valid
    col = lax.broadcasted_iota(jnp.int32, (_VP, _FO), 1)
    cnt = jnp.where((col == 0) & (row < _N_SH), 1.0, 0.0).astype(jnp.float32)
    return jnp.concatenate([cnt, lin], axis=1)


def _tc_pre_body(sh_ref, w_ref, b_ref, out_ref):
    lin = lax.dot_general(sh_ref[...], w_ref[...], (((1,), (1,)), ((), ())),
                          preferred_element_type=jnp.float32) + b_ref[...]
    out_ref[...] = _aug(lin)


def _combined_mean_tanh(acc):
    a = acc[0] + acc[1]
    cnt = a[:, 0:1]
    feat = a[:, _FO:_AW]
    return jnp.tanh(feat / jnp.maximum(cnt, 1.0))


def _tc_mid_body(acc_ref, w_ref, b_ref, out_ref):
    x2 = _combined_mean_tanh(acc_ref[...])
    lin = lax.dot_general(x2, w_ref[...], (((1,), (1,)), ((), ())),
                          preferred_element_type=jnp.float32) + b_ref[...]
    out_ref[...] = _aug(lin)


def _l2norm(x):
    n = jnp.sqrt(jnp.sum(x * x, axis=1, keepdims=True))
    return x / jnp.maximum(n, 1e-12)


def _kmeans(x, k):
    n = x.shape[0]
    cents = x[:k]
    ones = jnp.ones((n, 1), jnp.float32)
    kio = lax.broadcasted_iota(jnp.int32, (1, k), 1)

    def assign(cents):
        bd = jnp.full((n, 1), jnp.inf, jnp.float32)
        bi = jnp.zeros((n, 1), jnp.int32)
        for j in range(k):
            diff = x - cents[j:j + 1]
            dj = jnp.sum(diff * diff, axis=1, keepdims=True)
            upd = dj < bd
            bd = jnp.where(upd, dj, bd)
            bi = jnp.where(upd, j, bi)
        return bi

    for _ in range(4):
        bi = assign(cents)
        onehot = (bi == kio).astype(jnp.float32)
        s = lax.dot_general(onehot, x, (((0,), (0,)), ((), ())),
                            precision=lax.Precision.HIGHEST,
                            preferred_element_type=jnp.float32)
        c = lax.dot_general(onehot, ones, (((0,), (0,)), ((), ())),
                            precision=lax.Precision.HIGHEST,
                            preferred_element_type=jnp.float32)
        cents = s / jnp.maximum(c, 1.0)
    bi = assign(cents)
    onehot = (bi == kio).astype(jnp.float32)
    cents = _l2norm(cents)
    gathered = lax.dot_general(onehot, cents, (((1,), (0,)), ((), ())),
                               precision=lax.Precision.HIGHEST,
                               preferred_element_type=jnp.float32)
    return gathered


def _tc_final_body(acc_ref, sh_ref, presc_ref, mlpw_ref, mlpb_ref,
                   g_ref, bta_ref, out_ref):
    x6 = _combined_mean_tanh(acc_ref[...])
    sh = sh_ref[...]
    c_u = x6[:_NUM_U]
    c_i = x6[_NUM_U:_N_SH]
    p_u = sh[:_NUM_U]
    p_i = sh[_NUM_U:_N_SH]

    s_i = _l2norm(p_u)
    s_u = _l2norm(p_i)
    n_i = _kmeans(c_u, _K_U)
    n_u = _kmeans(c_i, _K_I)

    es = s_u + n_u
    eh = s_i + n_i

    presc = presc_ref[...]
    e_synd = lax.dot_general(presc, es, (((1,), (0,)), ((), ())),
                             preferred_element_type=jnp.float32)
    pre_sum = jnp.sum(presc, axis=1, keepdims=True)
    en = e_synd / pre_sum
    en = lax.dot_general(en, mlpw_ref[...], (((1,), (1,)), ((), ())),
                         preferred_element_type=jnp.float32) + mlpb_ref[...]
    mean = jnp.mean(en, axis=0, keepdims=True)
    var = jnp.mean((en - mean) * (en - mean), axis=0, keepdims=True)
    en = (en - mean) / jnp.sqrt(var + 1e-5) * g_ref[...] + bta_ref[...]
    en = jnp.maximum(en, 0.0)
    out_ref[...] = lax.dot_general(en, eh, (((1,), (1,)), ((), ())),
                                   preferred_element_type=jnp.float32)


def _tc_pre(sh_pad, w1, b1, interpret=False):
    return pl.pallas_call(
        _tc_pre_body,
        out_shape=jax.ShapeDtypeStruct((_VP, _AW), jnp.float32),
        interpret=interpret,
    )(sh_pad, w1, b1)


def _tc_mid(acc, w2, b2, interpret=False):
    return pl.pallas_call(
        _tc_mid_body,
        out_shape=jax.ShapeDtypeStruct((_VP, _AW), jnp.float32),
        interpret=interpret,
    )(acc, w2, b2)


def _tc_final(acc, sh_pad, presc, mlp_w, mlp_b, gamma, beta, interpret=False):
    return pl.pallas_call(
        _tc_final_body,
        out_shape=jax.ShapeDtypeStruct((1024, _NUM_U), jnp.float32),
        interpret=interpret,
    )(acc, sh_pad, presc, mlp_w, mlp_b, gamma, beta)


def kernel(x_SH, edge_index_SH, x_SS, edge_index_SS, x_HH, edge_index_HH,
           prescription, SH_table, W1, b1, W2, b2, W1h, b1h, W2h, b2h,
           mlp_W, mlp_b, bn_gamma, bn_beta):
    sh_pad = jnp.pad(SH_table.astype(jnp.float32), ((0, _VP - _N_SH), (0, 0)))
    npad = _E_PAD - edge_index_SH.shape[1]
    src = jnp.pad(edge_index_SH[0].astype(jnp.int32), (0, npad),
                  constant_values=_VP - 1).reshape(_NC * _NS, _RB, _EB)
    dst = jnp.pad(edge_index_SH[1].astype(jnp.int32), (0, npad),
                  constant_values=_VP - 1).reshape(_NC * _NS, _RB, _EB)
    zeros = jnp.zeros((_VP, _AW), jnp.float32)

    b1r = b1.reshape(1, _D).astype(jnp.float32)
    b2r = b2.reshape(1, _D).astype(jnp.float32)
    mlpb = mlp_b.reshape(1, _D).astype(jnp.float32)
    gamma = bn_gamma.reshape(1, _D).astype(jnp.float32)
    beta = bn_beta.reshape(1, _D).astype(jnp.float32)

    aug1 = _tc_pre(sh_pad, W1.astype(jnp.float32), b1r)
    acc1 = _sc_agg(aug1, src, dst, zeros)
    aug2 = _tc_mid(acc1, W2.astype(jnp.float32), b2r)
    acc2 = _sc_agg(aug2, src, dst, zeros)
    return _tc_final(acc2, sh_pad, prescription.astype(jnp.float32),
                     mlp_W.astype(jnp.float32), mlpb, gamma, beta)


# speedup vs baseline: 5.4907x; 2.5429x over previous
import functools

import jax
import jax.numpy as jnp
from jax import lax
from jax.experimental import pallas as pl
from jax.experimental.pallas import tpu as pltpu
from jax.experimental.pallas import tpu_sc as plsc

_NUM_U = 805
_NUM_I = 390
_N_SH = 1195
_D = 64
_K_U = 20
_K_I = 9

_VP = 1280
_AW = 128
_FO = 64
_NC = 2
_NS = 16
_EB = 128
_RB = 10
_E_PAD = _NC * _NS * _RB * _EB
_ROWS_PER_TILE = _VP // _NS


def _sc_agg_body(table_hbm, src_hbm, dst_hbm, zeros_hbm, out_hbm,
                 src_v, dst_v, rows0_v, rows1_v, stage_v, acc_sh,
                 sem0, sem1):
    c = lax.axis_index("c")
    s = lax.axis_index("s")
    wid = c * _NS + s
    row0 = s * _ROWS_PER_TILE
    bufs = (rows0_v, rows1_v)
    sems = (sem0, sem1)

    pltpu.sync_copy(zeros_hbm.at[pl.ds(row0, _ROWS_PER_TILE)], stage_v)
    pltpu.sync_copy(stage_v, acc_sh.at[pl.ds(row0, _ROWS_PER_TILE)])

    pltpu.sync_copy(src_hbm.at[wid], src_v)
    pltpu.sync_copy(dst_hbm.at[wid], dst_v)
    plsc.subcore_barrier()

    cps = [None] * _RB
    cps[0] = pltpu.async_copy(table_hbm.at[src_v.at[0]], bufs[0], sems[0])
    for r in range(_RB):
        if r + 1 < _RB:
            cps[r + 1] = pltpu.async_copy(
                table_hbm.at[src_v.at[r + 1]], bufs[(r + 1) % 2],
                sems[(r + 1) % 2])
        cps[r].wait()
        pltpu.sync_copy(bufs[r % 2], acc_sh.at[dst_v.at[r]], add=True)
    plsc.subcore_barrier()

    pltpu.sync_copy(acc_sh.at[pl.ds(row0, _ROWS_PER_TILE)], stage_v)
    pltpu.sync_copy(stage_v, out_hbm.at[c, pl.ds(row0, _ROWS_PER_TILE)])


@functools.cache
def _make_sc_agg():
    return pl.kernel(
        _sc_agg_body,
        out_type=jax.ShapeDtypeStruct((_NC, _VP, _AW), jnp.float32),
        mesh=plsc.VectorSubcoreMesh(core_axis_name="c", subcore_axis_name="s",
                                    num_cores=_NC, num_subcores=_NS),
        scratch_types=[
            pltpu.VMEM((_RB, _EB), jnp.int32),
            pltpu.VMEM((_RB, _EB), jnp.int32),
            pltpu.VMEM((_EB, _AW), jnp.float32),
            pltpu.VMEM((_EB, _AW), jnp.float32),
            pltpu.VMEM((_ROWS_PER_TILE, _AW), jnp.float32),
            pltpu.VMEM_SHARED((_VP, _AW), jnp.float32),
            pltpu.SemaphoreType.DMA,
            pltpu.SemaphoreType.DMA,
        ],
    )


def _sc_agg(table, src, dst, zeros):
    return _make_sc_agg()(table, src, dst, zeros)


def _aug(lin):
    row = lax.broadcasted_iota(jnp.int32, (_VP, 1), 0)
    valid = (row < _N_SH).astype(jnp.float32)
    lin = lin * valid
    col = lax.broadcasted_iota(jnp.int32, (_VP, _FO), 1)
    cnt = jnp.where((col == 0) & (row < _N_SH), 1.0, 0.0).astype(jnp.float32)
    return jnp.concatenate([cnt, lin], axis=1)


def _tc_pre_body(sh_ref, w_ref, b_ref, out_ref):
    lin = lax.dot_general(sh_ref[...], w_ref[...], (((1,), (1,)), ((), ())),
                          preferred_element_type=jnp.float32) + b_ref[...]
    out_ref[...] = _aug(lin)


def _combined_mean_tanh(acc):
    a = acc[0] + acc[1]
    cnt = a[:, 0:1]
    feat = a[:, _FO:_AW]
    return jnp.tanh(feat / jnp.maximum(cnt, 1.0))


def _tc_mid_body(acc_ref, w_ref, b_ref, out_ref):
    x2 = _combined_mean_tanh(acc_ref[...])
    lin = lax.dot_general(x2, w_ref[...], (((1,), (1,)), ((), ())),
                          preferred_element_type=jnp.float32) + b_ref[...]
    out_ref[...] = _aug(lin)


def _l2norm(x):
    n = jnp.sqrt(jnp.sum(x * x, axis=1, keepdims=True))
    return x / jnp.maximum(n, 1e-12)


def _kmeans(x, k):
    n = x.shape[0]
    cents = x[:k]
    ones = jnp.ones((n, 1), jnp.float32)
    kio = lax.broadcasted_iota(jnp.int32, (1, k), 1)

    def assign(cents):
        bd = jnp.full((n, 1), jnp.inf, jnp.float32)
        bi = jnp.zeros((n, 1), jnp.int32)
        for j in range(k):
            diff = x - cents[j:j + 1]
            dj = jnp.sum(diff * diff, axis=1, keepdims=True)
            upd = dj < bd
            bd = jnp.where(upd, dj, bd)
            bi = jnp.where(upd, j, bi)
        return bi

    for _ in range(4):
        bi = assign(cents)
        onehot = (bi == kio).astype(jnp.float32)
        s = lax.dot_general(onehot, x, (((0,), (0,)), ((), ())),
                            precision=lax.Precision.HIGHEST,
                            preferred_element_type=jnp.float32)
        c = lax.dot_general(onehot, ones, (((0,), (0,)), ((), ())),
                            precision=lax.Precision.HIGHEST,
                            preferred_element_type=jnp.float32)
        cents = s / jnp.maximum(c, 1.0)
    bi = assign(cents)
    onehot = (bi == kio).astype(jnp.float32)
    cents = _l2norm(cents)
    gathered = lax.dot_general(onehot, cents, (((1,), (0,)), ((), ())),
                               precision=lax.Precision.HIGHEST,
                               preferred_element_type=jnp.float32)
    return gathered


def _tc_final_body(acc_ref, sh_ref, presc_ref, mlpw_ref, mlpb_ref,
                   g_ref, bta_ref, out_ref):
    x6 = _combined_mean_tanh(acc_ref[...])
    sh = sh_ref[...]
    c_u = x6[:_NUM_U]
    c_i = x6[_NUM_U:_N_SH]
    p_u = sh[:_NUM_U]
    p_i = sh[_NUM_U:_N_SH]

    s_i = _l2norm(p_u)
    s_u = _l2norm(p_i)
    n_i = _kmeans(c_u, _K_U)
    n_u = _kmeans(c_i, _K_I)

    es = s_u + n_u
    eh = s_i + n_i

    presc = presc_ref[...]
    e_synd = lax.dot_general(presc, es, (((1,), (0,)), ((), ())),
                             preferred_element_type=jnp.float32)
    pre_sum = jnp.sum(presc, axis=1, keepdims=True)
    en = e_synd / pre_sum
    en = lax.dot_general(en, mlpw_ref[...], (((1,), (1,)), ((), ())),
                         preferred_element_type=jnp.float32) + mlpb_ref[...]
    mean = jnp.mean(en, axis=0, keepdims=True)
    var = jnp.mean((en - mean) * (en - mean), axis=0, keepdims=True)
    en = (en - mean) / jnp.sqrt(var + 1e-5) * g_ref[...] + bta_ref[...]
    en = jnp.maximum(en, 0.0)
    out_ref[...] = lax.dot_general(en, eh, (((1,), (1,)), ((), ())),
                                   preferred_element_type=jnp.float32)


def _tc_pre(sh_pad, w1, b1, interpret=False):
    return pl.pallas_call(
        _tc_pre_body,
        out_shape=jax.ShapeDtypeStruct((_VP, _AW), jnp.float32),
        interpret=interpret,
    )(sh_pad, w1, b1)


def _tc_mid(acc, w2, b2, interpret=False):
    return pl.pallas_call(
        _tc_mid_body,
        out_shape=jax.ShapeDtypeStruct((_VP, _AW), jnp.float32),
        interpret=interpret,
    )(acc, w2, b2)


def _tc_final(acc, sh_pad, presc, mlp_w, mlp_b, gamma, beta, interpret=False):
    return pl.pallas_call(
        _tc_final_body,
        out_shape=jax.ShapeDtypeStruct((1024, _NUM_U), jnp.float32),
        interpret=interpret,
    )(acc, sh_pad, presc, mlp_w, mlp_b, gamma, beta)


def kernel(x_SH, edge_index_SH, x_SS, edge_index_SS, x_HH, edge_index_HH,
           prescription, SH_table, W1, b1, W2, b2, W1h, b1h, W2h, b2h,
           mlp_W, mlp_b, bn_gamma, bn_beta):
    sh_pad = jnp.pad(SH_table.astype(jnp.float32), ((0, _VP - _N_SH), (0, 0)))
    ne = edge_index_SH.shape[1]
    npad = _E_PAD - ne
    src_e = edge_index_SH[0].astype(jnp.int32)
    src = jnp.concatenate([src_e, src_e[:npad]]).reshape(_NC * _NS, _RB, _EB)
    dst = jnp.pad(edge_index_SH[1].astype(jnp.int32), (0, npad),
                  constant_values=_VP - 1).reshape(_NC * _NS, _RB, _EB)
    zeros = jnp.zeros((_VP, _AW), jnp.float32)

    b1r = b1.reshape(1, _D).astype(jnp.float32)
    b2r = b2.reshape(1, _D).astype(jnp.float32)
    mlpb = mlp_b.reshape(1, _D).astype(jnp.float32)
    gamma = bn_gamma.reshape(1, _D).astype(jnp.float32)
    beta = bn_beta.reshape(1, _D).astype(jnp.float32)

    aug1 = _tc_pre(sh_pad, W1.astype(jnp.float32), b1r)
    acc1 = _sc_agg(aug1, src, dst, zeros)
    aug2 = _tc_mid(acc1, W2.astype(jnp.float32), b2r)
    acc2 = _sc_agg(aug2, src, dst, zeros)
    return _tc_final(acc2, sh_pad, prescription.astype(jnp.float32),
                     mlp_W.astype(jnp.float32), mlpb, gamma, beta)


# speedup vs baseline: 7.9183x; 1.4421x over previous
import functools

import jax
import jax.numpy as jnp
from jax import lax
from jax.experimental import pallas as pl
from jax.experimental.pallas import tpu as pltpu
from jax.experimental.pallas import tpu_sc as plsc

_NUM_U = 805
_NUM_I = 390
_N_SH = 1195
_D = 64
_K_U = 20
_K_I = 9

_VP = 1280
_AW = 128
_FO = 64
_NC = 2
_NS = 16
_EB = 128
_RB = 10
_E_PAD = _NC * _NS * _RB * _EB
_ROWS_PER_TILE = _VP // _NS


def _sc_agg_body(table_hbm, src_hbm, dst_hbm, zeros_hbm, out_hbm,
                 src_v, dst_v, rows0_v, rows1_v, stage_v, acc_sh,
                 sem0, sem1):
    c = lax.axis_index("c")
    s = lax.axis_index("s")
    wid = c * _NS + s
    row0 = s * _ROWS_PER_TILE
    bufs = (rows0_v, rows1_v)
    sems = (sem0, sem1)

    pltpu.sync_copy(zeros_hbm.at[pl.ds(row0, _ROWS_PER_TILE)], stage_v)
    pltpu.sync_copy(stage_v, acc_sh.at[pl.ds(row0, _ROWS_PER_TILE)])

    pltpu.sync_copy(src_hbm.at[wid], src_v)
    pltpu.sync_copy(dst_hbm.at[wid], dst_v)
    plsc.subcore_barrier()

    cps = [None] * _RB
    cps[0] = pltpu.async_copy(table_hbm.at[src_v.at[0]], bufs[0], sems[0])
    for r in range(_RB):
        if r + 1 < _RB:
            cps[r + 1] = pltpu.async_copy(
                table_hbm.at[src_v.at[r + 1]], bufs[(r + 1) % 2],
                sems[(r + 1) % 2])
        cps[r].wait()
        pltpu.sync_copy(bufs[r % 2], acc_sh.at[dst_v.at[r]], add=True)
    plsc.subcore_barrier()

    pltpu.sync_copy(acc_sh.at[pl.ds(row0, _ROWS_PER_TILE)], stage_v)
    pltpu.sync_copy(stage_v, out_hbm.at[c, pl.ds(row0, _ROWS_PER_TILE)])


@functools.cache
def _make_sc_agg():
    return pl.kernel(
        _sc_agg_body,
        out_type=jax.ShapeDtypeStruct((_NC, _VP, _AW), jnp.float32),
        mesh=plsc.VectorSubcoreMesh(core_axis_name="c", subcore_axis_name="s",
                                    num_cores=_NC, num_subcores=_NS),
        scratch_types=[
            pltpu.VMEM((_RB, _EB), jnp.int32),
            pltpu.VMEM((_RB, _EB), jnp.int32),
            pltpu.VMEM((_EB, _AW), jnp.float32),
            pltpu.VMEM((_EB, _AW), jnp.float32),
            pltpu.VMEM((_ROWS_PER_TILE, _AW), jnp.float32),
            pltpu.VMEM_SHARED((_VP, _AW), jnp.float32),
            pltpu.SemaphoreType.DMA,
            pltpu.SemaphoreType.DMA,
        ],
    )


def _sc_agg(table, src, dst, zeros):
    return _make_sc_agg()(table, src, dst, zeros)


def _aug(lin):
    row = lax.broadcasted_iota(jnp.int32, (_VP, 1), 0)
    valid = (row < _N_SH).astype(jnp.float32)
    lin = lin * valid
    col = lax.broadcasted_iota(jnp.int32, (_VP, _FO), 1)
    cnt = jnp.where((col == 0) & (row < _N_SH), 1.0, 0.0).astype(jnp.float32)
    return jnp.concatenate([cnt, lin], axis=1)


def _tc_pre_body(sh_ref, w_ref, b_ref, out_ref):
    lin = lax.dot_general(sh_ref[...], w_ref[...], (((1,), (1,)), ((), ())),
                          preferred_element_type=jnp.float32) + b_ref[...]
    out_ref[...] = _aug(lin)


def _combined_mean_tanh(acc):
    a = acc[0] + acc[1]
    cnt = a[:, 0:1]
    feat = a[:, _FO:_AW]
    return jnp.tanh(feat / jnp.maximum(cnt, 1.0))


def _tc_mid_body(acc_ref, w_ref, b_ref, out_ref):
    x2 = _combined_mean_tanh(acc_ref[...])
    lin = lax.dot_general(x2, w_ref[...], (((1,), (1,)), ((), ())),
                          preferred_element_type=jnp.float32) + b_ref[...]
    out_ref[...] = _aug(lin)


def _l2norm_cols(xT):
    n = jnp.sqrt(jnp.sum(xT * xT, axis=0, keepdims=True))
    return xT / jnp.maximum(n, 1e-12)


def _kmeans_t(xT, k):
    n = xT.shape[1]
    centsT = xT[:, :k]
    jio = lax.broadcasted_iota(jnp.int32, (k, 1), 0)
    ones = jnp.ones((1, n), jnp.float32)

    def assign(centsT):
        bd = jnp.full((1, n), jnp.inf, jnp.float32)
        bi = jnp.zeros((1, n), jnp.int32)
        for j in range(k):
            diff = xT - centsT[:, j:j + 1]
            dj = jnp.sum(diff * diff, axis=0, keepdims=True)
            upd = dj < bd
            bd = jnp.where(upd, dj, bd)
            bi = jnp.where(upd, j, bi)
        return bi

    for _ in range(4):
        bi = assign(centsT)
        onehotT = (bi == jio).astype(jnp.float32)
        sT = lax.dot_general(xT, onehotT, (((1,), (1,)), ((), ())),
                             precision=lax.Precision.HIGHEST,
                             preferred_element_type=jnp.float32)
        cnt = lax.dot_general(ones, onehotT, (((1,), (1,)), ((), ())),
                              precision=lax.Precision.HIGHEST,
                              preferred_element_type=jnp.float32)
        centsT = sT / jnp.maximum(cnt, 1.0)
    bi = assign(centsT)
    onehotT = (bi == jio).astype(jnp.float32)
    centsT = _l2norm_cols(centsT)
    gatheredT = lax.dot_general(centsT, onehotT, (((1,), (0,)), ((), ())),
                                precision=lax.Precision.HIGHEST,
                                preferred_element_type=jnp.float32)
    return gatheredT


def _tc_final_body(acc_ref, sh_ref, presc_ref, mlpw_ref, mlpb_ref,
                   g_ref, bta_ref, out_ref):
    x6 = _combined_mean_tanh(acc_ref[...])
    x6T = x6.T
    shT = sh_ref[...].T

    s_iT = _l2norm_cols(shT[:, :_NUM_U])
    s_uT = _l2norm_cols(shT[:, _NUM_U:_N_SH])
    n_iT = _kmeans_t(x6T[:, :_NUM_U], _K_U)
    n_uT = _kmeans_t(x6T[:, _NUM_U:_N_SH], _K_I)

    esT = s_uT + n_uT
    ehT = s_iT + n_iT

    presc = presc_ref[...]
    e_synd = lax.dot_general(presc, esT, (((1,), (1,)), ((), ())),
                             preferred_element_type=jnp.float32)
    pre_sum = jnp.sum(presc, axis=1, keepdims=True)
    en = e_synd / pre_sum
    en = lax.dot_general(en, mlpw_ref[...], (((1,), (1,)), ((), ())),
                         preferred_element_type=jnp.float32) + mlpb_ref[...]
    mean = jnp.mean(en, axis=0, keepdims=True)
    var = jnp.mean((en - mean) * (en - mean), axis=0, keepdims=True)
    en = (en - mean) / jnp.sqrt(var + 1e-5) * g_ref[...] + bta_ref[...]
    en = jnp.maximum(en, 0.0)
    out_ref[...] = lax.dot_general(en, ehT, (((1,), (0,)), ((), ())),
                                   preferred_element_type=jnp.float32)


def _tc_pre(sh_pad, w1, b1, interpret=False):
    return pl.pallas_call(
        _tc_pre_body,
        out_shape=jax.ShapeDtypeStruct((_VP, _AW), jnp.float32),
        interpret=interpret,
    )(sh_pad, w1, b1)


def _tc_mid(acc, w2, b2, interpret=False):
    return pl.pallas_call(
        _tc_mid_body,
        out_shape=jax.ShapeDtypeStruct((_VP, _AW), jnp.float32),
        interpret=interpret,
    )(acc, w2, b2)


def _tc_final(acc, sh_pad, presc, mlp_w, mlp_b, gamma, beta, interpret=False):
    return pl.pallas_call(
        _tc_final_body,
        out_shape=jax.ShapeDtypeStruct((1024, _NUM_U), jnp.float32),
        interpret=interpret,
    )(acc, sh_pad, presc, mlp_w, mlp_b, gamma, beta)


def kernel(x_SH, edge_index_SH, x_SS, edge_index_SS, x_HH, edge_index_HH,
           prescription, SH_table, W1, b1, W2, b2, W1h, b1h, W2h, b2h,
           mlp_W, mlp_b, bn_gamma, bn_beta):
    sh_pad = jnp.pad(SH_table.astype(jnp.float32), ((0, _VP - _N_SH), (0, 0)))
    ne = edge_index_SH.shape[1]
    npad = _E_PAD - ne
    src_e = edge_index_SH[0].astype(jnp.int32)
    src = jnp.concatenate([src_e, src_e[:npad]]).reshape(_NC * _NS, _RB, _EB)
    dst = jnp.pad(edge_index_SH[1].astype(jnp.int32), (0, npad),
                  constant_values=_VP - 1).reshape(_NC * _NS, _RB, _EB)
    zeros = jnp.zeros((_VP, _AW), jnp.float32)

    b1r = b1.reshape(1, _D).astype(jnp.float32)
    b2r = b2.reshape(1, _D).astype(jnp.float32)
    mlpb = mlp_b.reshape(1, _D).astype(jnp.float32)
    gamma = bn_gamma.reshape(1, _D).astype(jnp.float32)
    beta = bn_beta.reshape(1, _D).astype(jnp.float32)

    aug1 = _tc_pre(sh_pad, W1.astype(jnp.float32), b1r)
    acc1 = _sc_agg(aug1, src, dst, zeros)
    aug2 = _tc_mid(acc1, W2.astype(jnp.float32), b2r)
    acc2 = _sc_agg(aug2, src, dst, zeros)
    return _tc_final(acc2, sh_pad, prescription.astype(jnp.float32),
                     mlp_W.astype(jnp.float32), mlpb, gamma, beta)


# speedup vs baseline: 7.9596x; 1.0052x over previous
import functools

import jax
import jax.numpy as jnp
from jax import lax
from jax.experimental import pallas as pl
from jax.experimental.pallas import tpu as pltpu
from jax.experimental.pallas import tpu_sc as plsc

_NUM_U = 805
_NUM_I = 390
_N_SH = 1195
_D = 64
_K_U = 20
_K_I = 9

_VP = 1280
_AW = 128
_FO = 64
_NC = 2
_NS = 16
_EB = 128
_RB = 10
_E_PAD = _NC * _NS * _RB * _EB
_ROWS_PER_TILE = _VP // _NS


_NBUF = 4
_GLAG = 2


def _sc_agg_body(table_hbm, src_hbm, dst_hbm, zeros_hbm, out_hbm,
                 src_v, dst_v, bufs_v, stage_v, acc_sh,
                 gsem0, gsem1, ssem0, ssem1):
    c = lax.axis_index("c")
    s = lax.axis_index("s")
    wid = c * _NS + s
    row0 = s * _ROWS_PER_TILE
    gsems = (gsem0, gsem1)
    ssems = (ssem0, ssem1)

    pltpu.sync_copy(zeros_hbm.at[pl.ds(row0, _ROWS_PER_TILE)], stage_v)
    pltpu.sync_copy(stage_v, acc_sh.at[pl.ds(row0, _ROWS_PER_TILE)])

    pltpu.sync_copy(src_hbm.at[wid], src_v)
    pltpu.sync_copy(dst_hbm.at[wid], dst_v)
    plsc.subcore_barrier()

    gcp = [None] * _RB
    scp = [None] * _RB
    for r in range(_GLAG):
        gcp[r] = pltpu.async_copy(table_hbm.at[src_v.at[r]],
                                  bufs_v.at[r % _NBUF], gsems[r % _GLAG])
    for r in range(_RB):
        gcp[r].wait()
        nr = r + _GLAG
        if nr < _RB:
            if nr - _NBUF >= 0:
                scp[nr - _NBUF].wait()
            gcp[nr] = pltpu.async_copy(table_hbm.at[src_v.at[nr]],
                                       bufs_v.at[nr % _NBUF],
                                       gsems[nr % _GLAG])
        scp[r] = pltpu.async_copy(bufs_v.at[r % _NBUF],
                                  acc_sh.at[dst_v.at[r]],
                                  ssems[r % (_NBUF - _GLAG)], add=True)
    for r in range(_RB - _NBUF, _RB):
        if r >= 0:
            scp[r].wait()
    plsc.subcore_barrier()

    pltpu.sync_copy(acc_sh.at[pl.ds(row0, _ROWS_PER_TILE)], stage_v)
    pltpu.sync_copy(stage_v, out_hbm.at[c, pl.ds(row0, _ROWS_PER_TILE)])


@functools.cache
def _make_sc_agg():
    return pl.kernel(
        _sc_agg_body,
        out_type=jax.ShapeDtypeStruct((_NC, _VP, _AW), jnp.float32),
        mesh=plsc.VectorSubcoreMesh(core_axis_name="c", subcore_axis_name="s",
                                    num_cores=_NC, num_subcores=_NS),
        scratch_types=[
            pltpu.VMEM((_RB, _EB), jnp.int32),
            pltpu.VMEM((_RB, _EB), jnp.int32),
            pltpu.VMEM((_NBUF, _EB, _AW), jnp.float32),
            pltpu.VMEM((_ROWS_PER_TILE, _AW), jnp.float32),
            pltpu.VMEM_SHARED((_VP, _AW), jnp.float32),
            pltpu.SemaphoreType.DMA,
            pltpu.SemaphoreType.DMA,
            pltpu.SemaphoreType.DMA,
            pltpu.SemaphoreType.DMA,
        ],
    )


def _sc_agg(table, src, dst, zeros):
    return _make_sc_agg()(table, src, dst, zeros)


def _aug(lin):
    row = lax.broadcasted_iota(jnp.int32, (_VP, 1), 0)
    valid = (row < _N_SH).astype(jnp.float32)
    lin = lin * valid
    col = lax.broadcasted_iota(jnp.int32, (_VP, _FO), 1)
    cnt = jnp.where((col == 0) & (row < _N_SH), 1.0, 0.0).astype(jnp.float32)
    return jnp.concatenate([cnt, lin], axis=1)


def _tc_pre_body(sh_ref, w_ref, b_ref, out_ref):
    lin = lax.dot_general(sh_ref[...], w_ref[...], (((1,), (1,)), ((), ())),
                          preferred_element_type=jnp.float32) + b_ref[...]
    out_ref[...] = _aug(lin)


def _combined_mean_tanh(acc):
    a = acc[0] + acc[1]
    cnt = a[:, 0:1]
    feat = a[:, _FO:_AW]
    return jnp.tanh(feat / jnp.maximum(cnt, 1.0))


def _tc_mid_body(acc_ref, w_ref, b_ref, out_ref):
    x2 = _combined_mean_tanh(acc_ref[...])
    lin = lax.dot_general(x2, w_ref[...], (((1,), (1,)), ((), ())),
                          preferred_element_type=jnp.float32) + b_ref[...]
    out_ref[...] = _aug(lin)


def _l2norm_cols(xT):
    n = jnp.sqrt(jnp.sum(xT * xT, axis=0, keepdims=True))
    return xT / jnp.maximum(n, 1e-12)


def _kmeans_t(xT, k):
    n = xT.shape[1]
    centsT = xT[:, :k]
    jio = lax.broadcasted_iota(jnp.int32, (k, 1), 0)
    ones = jnp.ones((1, n), jnp.float32)

    def assign(centsT):
        bd = jnp.full((1, n), jnp.inf, jnp.float32)
        bi = jnp.zeros((1, n), jnp.int32)
        for j in range(k):
            diff = xT - centsT[:, j:j + 1]
            dj = jnp.sum(diff * diff, axis=0, keepdims=True)
            upd = dj < bd
            bd = jnp.where(upd, dj, bd)
            bi = jnp.where(upd, j, bi)
        return bi

    for _ in range(4):
        bi = assign(centsT)
        onehotT = (bi == jio).astype(jnp.float32)
        sT = lax.dot_general(xT, onehotT, (((1,), (1,)), ((), ())),
                             precision=lax.Precision.HIGHEST,
                             preferred_element_type=jnp.float32)
        cnt = lax.dot_general(ones, onehotT, (((1,), (1,)), ((), ())),
                              precision=lax.Precision.HIGHEST,
                              preferred_element_type=jnp.float32)
        centsT = sT / jnp.maximum(cnt, 1.0)
    bi = assign(centsT)
    onehotT = (bi == jio).astype(jnp.float32)
    centsT = _l2norm_cols(centsT)
    gatheredT = lax.dot_general(centsT, onehotT, (((1,), (0,)), ((), ())),
                                precision=lax.Precision.HIGHEST,
                                preferred_element_type=jnp.float32)
    return gatheredT


def _tc_final_body(acc_ref, sh_ref, presc_ref, mlpw_ref, mlpb_ref,
                   g_ref, bta_ref, out_ref):
    x6 = _combined_mean_tanh(acc_ref[...])
    x6T = x6.T
    shT = sh_ref[...].T

    s_iT = _l2norm_cols(shT[:, :_NUM_U])
    s_uT = _l2norm_cols(shT[:, _NUM_U:_N_SH])
    n_iT = _kmeans_t(x6T[:, :_NUM_U], _K_U)
    n_uT = _kmeans_t(x6T[:, _NUM_U:_N_SH], _K_I)

    esT = s_uT + n_uT
    ehT = s_iT + n_iT

    presc = presc_ref[...]
    e_synd = lax.dot_general(presc, esT, (((1,), (1,)), ((), ())),
                             preferred_element_type=jnp.float32)
    pre_sum = jnp.sum(presc, axis=1, keepdims=True)
    en = e_synd / pre_sum
    en = lax.dot_general(en, mlpw_ref[...], (((1,), (1,)), ((), ())),
                         preferred_element_type=jnp.float32) + mlpb_ref[...]
    mean = jnp.mean(en, axis=0, keepdims=True)
    var = jnp.mean((en - mean) * (en - mean), axis=0, keepdims=True)
    en = (en - mean) / jnp.sqrt(var + 1e-5) * g_ref[...] + bta_ref[...]
    en = jnp.maximum(en, 0.0)
    out_ref[...] = lax.dot_general(en, ehT, (((1,), (0,)), ((), ())),
                                   preferred_element_type=jnp.float32)


def _tc_pre(sh_pad, w1, b1, interpret=False):
    return pl.pallas_call(
        _tc_pre_body,
        out_shape=jax.ShapeDtypeStruct((_VP, _AW), jnp.float32),
        interpret=interpret,
    )(sh_pad, w1, b1)


def _tc_mid(acc, w2, b2, interpret=False):
    return pl.pallas_call(
        _tc_mid_body,
        out_shape=jax.ShapeDtypeStruct((_VP, _AW), jnp.float32),
        interpret=interpret,
    )(acc, w2, b2)


def _tc_final(acc, sh_pad, presc, mlp_w, mlp_b, gamma, beta, interpret=False):
    return pl.pallas_call(
        _tc_final_body,
        out_shape=jax.ShapeDtypeStruct((1024, _NUM_U), jnp.float32),
        interpret=interpret,
    )(acc, sh_pad, presc, mlp_w, mlp_b, gamma, beta)


def kernel(x_SH, edge_index_SH, x_SS, edge_index_SS, x_HH, edge_index_HH,
           prescription, SH_table, W1, b1, W2, b2, W1h, b1h, W2h, b2h,
           mlp_W, mlp_b, bn_gamma, bn_beta):
    sh_pad = jnp.pad(SH_table.astype(jnp.float32), ((0, _VP - _N_SH), (0, 0)))
    ne = edge_index_SH.shape[1]
    npad = _E_PAD - ne
    src_e = edge_index_SH[0].astype(jnp.int32)
    src = jnp.concatenate([src_e, src_e[:npad]]).reshape(_NC * _NS, _RB, _EB)
    dst = jnp.pad(edge_index_SH[1].astype(jnp.int32), (0, npad),
                  constant_values=_VP - 1).reshape(_NC * _NS, _RB, _EB)
    zeros = jnp.zeros((_VP, _AW), jnp.float32)

    b1r = b1.reshape(1, _D).astype(jnp.float32)
    b2r = b2.reshape(1, _D).astype(jnp.float32)
    mlpb = mlp_b.reshape(1, _D).astype(jnp.float32)
    gamma = bn_gamma.reshape(1, _D).astype(jnp.float32)
    beta = bn_beta.reshape(1, _D).astype(jnp.float32)

    aug1 = _tc_pre(sh_pad, W1.astype(jnp.float32), b1r)
    acc1 = _sc_agg(aug1, src, dst, zeros)
    aug2 = _tc_mid(acc1, W2.astype(jnp.float32), b2r)
    acc2 = _sc_agg(aug2, src, dst, zeros)
    return _tc_final(acc2, sh_pad, prescription.astype(jnp.float32),
                     mlp_W.astype(jnp.float32), mlpb, gamma, beta)
